# baseline (speedup 1.0000x reference)
import functools

import jax
import jax.numpy as jnp
from jax import lax
from jax.experimental import pallas as pl
from jax.experimental.pallas import tpu as pltpu
from jax.experimental.pallas import tpu_sc as plsc

EMBED = 64
N_BINS = 256
_info = plsc.get_sparse_core_info()
NC, NS = _info.num_cores, _info.num_subcores
NW = NC * NS


def _make_sc_lookup(B: int, C: int):
    assert B % (NW * C) == 0
    b_per_w = B // NW
    chunks = b_per_w // C
    mesh = plsc.VectorSubcoreMesh(core_axis_name="c", subcore_axis_name="s")

    @functools.partial(
        pl.kernel,
        out_type=jax.ShapeDtypeStruct((B * EMBED,), jnp.float32),
        mesh=mesh,
        scratch_types=[
            pltpu.VMEM((N_BINS * EMBED,), jnp.float32),
            pltpu.VMEM((2, C), jnp.int32),
            pltpu.VMEM((2, C * EMBED), jnp.float32),
            pltpu.SemaphoreType.DMA((2,)),
            pltpu.SemaphoreType.DMA((2,)),
        ],
        compiler_params=pltpu.CompilerParams(use_tc_tiling_on_sc=False,
                                             needs_layout_passes=False),
    )
    def sc_lookup(x_hbm, table_hbm, out_hbm, table_v, idx_v, rows_v,
                  sem_idx, sem_out):
        wid = lax.axis_index("s") * NC + lax.axis_index("c")
        base0 = wid * b_per_w
        lane = lax.iota(jnp.int32, 16)

        pltpu.sync_copy(table_hbm, table_v)

        pltpu.async_copy(x_hbm.at[pl.ds(base0, C)], idx_v.at[0],
                         sem_idx.at[0])

        @pl.loop(0, chunks)
        def _chunk(c):
            b = c % 2
            nb = 1 - b

            @pl.when(c + 1 < chunks)
            def _prefetch():
                nbase = base0 + (c + 1) * C
                pltpu.async_copy(x_hbm.at[pl.ds(nbase, C)], idx_v.at[nb],
                                 sem_idx.at[nb])

            pltpu.make_async_copy(x_hbm.at[pl.ds(base0, C)], idx_v.at[b],
                                  sem_idx.at[b]).wait()

            @pl.when(c >= 2)
            def _drain():
                obase = (base0 + (c - 2) * C) * EMBED
                pltpu.make_async_copy(rows_v.at[b],
                                      out_hbm.at[pl.ds(obase, C * EMBED)],
                                      sem_out.at[b]).wait()

            rows_b = rows_v.at[b]

            @pl.loop(0, C // 16)
            def _i(i):
                idx16 = idx_v[b, pl.ds(i * 16, 16)]
                tpos = idx16 * EMBED
                opos = (lane + i * 16) * EMBED

                @pl.loop(0, EMBED, unroll=16)
                def _j(j):
                    vals = plsc.load_gather(table_v, [tpos + j])
                    plsc.store_scatter(rows_b, [opos + j], vals)

            obase = (base0 + c * C) * EMBED
            pltpu.async_copy(rows_b, out_hbm.at[pl.ds(obase, C * EMBED)],
                             sem_out.at[b])

        @pl.loop(0, 2)
        def _tail(t):
            c = chunks - 2 + t
            b = c % 2
            obase = (base0 + c * C) * EMBED
            pltpu.make_async_copy(rows_v.at[b],
                                  out_hbm.at[pl.ds(obase, C * EMBED)],
                                  sem_out.at[b]).wait()

    return sc_lookup


def kernel(x, table):
    lead = x.shape[:-1]
    k = x.shape[-1]
    B = 1
    for s in x.shape:
        B *= s
    xf = x.reshape(B).astype(jnp.int32)
    out = _make_sc_lookup(B, 512)(xf, table.reshape(-1))
    return out.reshape(lead + (k * EMBED,))


# speedup vs baseline: 1.2197x; 1.2197x over previous
import functools

import jax
import jax.numpy as jnp
from jax import lax
from jax.experimental import pallas as pl
from jax.experimental.pallas import tpu as pltpu
from jax.experimental.pallas import tpu_sc as plsc

EMBED = 64
N_BINS = 256
_info = plsc.get_sparse_core_info()
NC, NS = _info.num_cores, _info.num_subcores
NW = NC * NS


def _make_sc_lookup(B: int, C: int):
    assert B % (NW * C) == 0
    b_per_w = B // NW
    chunks = b_per_w // C
    mesh = plsc.VectorSubcoreMesh(core_axis_name="c", subcore_axis_name="s")

    @functools.partial(
        pl.kernel,
        out_type=jax.ShapeDtypeStruct((B * EMBED,), jnp.float32),
        mesh=mesh,
        scratch_types=[
            pltpu.VMEM((N_BINS * EMBED,), jnp.float32),
            pltpu.VMEM((2, C), jnp.int32),
            pltpu.VMEM((2, C * EMBED), jnp.float32),
            pltpu.SemaphoreType.DMA((2,)),
            pltpu.SemaphoreType.DMA((2,)),
        ],
        compiler_params=pltpu.CompilerParams(use_tc_tiling_on_sc=False,
                                             needs_layout_passes=False),
    )
    def sc_lookup(x_hbm, table_hbm, out_hbm, table_v, idx_v, rows_v,
                  sem_idx, sem_out):
        wid = lax.axis_index("s") * NC + lax.axis_index("c")
        base0 = wid * b_per_w
        lane = lax.iota(jnp.int32, 16)

        pltpu.sync_copy(table_hbm, table_v)

        pltpu.async_copy(x_hbm.at[pl.ds(base0, C)], idx_v.at[0],
                         sem_idx.at[0])

        @pl.loop(0, chunks)
        def _chunk(c):
            b = c % 2
            nb = 1 - b

            @pl.when(c + 1 < chunks)
            def _prefetch():
                nbase = base0 + (c + 1) * C
                pltpu.async_copy(x_hbm.at[pl.ds(nbase, C)], idx_v.at[nb],
                                 sem_idx.at[nb])

            pltpu.make_async_copy(x_hbm.at[pl.ds(base0, C)], idx_v.at[b],
                                  sem_idx.at[b]).wait()

            @pl.when(c >= 2)
            def _drain():
                obase = (base0 + (c - 2) * C) * EMBED
                pltpu.make_async_copy(rows_v.at[b],
                                      out_hbm.at[pl.ds(obase, C * EMBED)],
                                      sem_out.at[b]).wait()

            rows_b = rows_v.at[b]

            @plsc.parallel_loop(0, C // 16, unroll=2)
            def _i(i):
                idx16 = idx_v[b, pl.ds(i * 16, 16)]
                tpos = idx16 * EMBED
                opos = (lane + i * 16) * EMBED
                for j in range(EMBED):
                    vals = plsc.load_gather(table_v, [tpos + j])
                    plsc.store_scatter(rows_b, [opos + j], vals)

            obase = (base0 + c * C) * EMBED
            pltpu.async_copy(rows_b, out_hbm.at[pl.ds(obase, C * EMBED)],
                             sem_out.at[b])

        @pl.loop(0, 2)
        def _tail(t):
            c = chunks - 2 + t
            b = c % 2
            obase = (base0 + c * C) * EMBED
            pltpu.make_async_copy(rows_v.at[b],
                                  out_hbm.at[pl.ds(obase, C * EMBED)],
                                  sem_out.at[b]).wait()

    return sc_lookup


def kernel(x, table):
    lead = x.shape[:-1]
    k = x.shape[-1]
    B = 1
    for s in x.shape:
        B *= s
    xf = x.reshape(B).astype(jnp.int32)
    out = _make_sc_lookup(B, 512)(xf, table.reshape(-1))
    return out.reshape(lead + (k * EMBED,))


# speedup vs baseline: 4.1539x; 3.4057x over previous
import functools

import jax
import jax.numpy as jnp
from jax import lax
from jax.experimental import pallas as pl
from jax.experimental.pallas import tpu as pltpu
from jax.experimental.pallas import tpu_sc as plsc

EMBED = 64
N_BINS = 256
_info = plsc.get_sparse_core_info()
NC, NS = _info.num_cores, _info.num_subcores
NW = NC * NS


def _make_sc_lookup(B: int, C: int):
    assert B % (NW * C) == 0
    b_per_w = B // NW
    chunks = b_per_w // C
    mesh = plsc.VectorSubcoreMesh(core_axis_name="c", subcore_axis_name="s")

    @functools.partial(
        pl.kernel,
        out_type=jax.ShapeDtypeStruct((B * EMBED,), jnp.float32),
        mesh=mesh,
        scratch_types=[
            pltpu.VMEM((N_BINS * EMBED,), jnp.float32),
            pltpu.VMEM((2, C), jnp.int32),
            pltpu.VMEM((2, C * EMBED), jnp.float32),
            pltpu.SemaphoreType.DMA((2,)),
            pltpu.SemaphoreType.DMA((2,)),
        ],
        compiler_params=pltpu.CompilerParams(use_tc_tiling_on_sc=False,
                                             needs_layout_passes=False),
    )
    def sc_lookup(x_hbm, table_hbm, out_hbm, table_v, idx_v, rows_v,
                  sem_idx, sem_out):
        wid = lax.axis_index("s") * NC + lax.axis_index("c")
        base0 = wid * b_per_w

        pltpu.sync_copy(table_hbm, table_v)

        pltpu.async_copy(x_hbm.at[pl.ds(base0, C)], idx_v.at[0],
                         sem_idx.at[0])

        @pl.loop(0, chunks)
        def _chunk(c):
            b = c % 2
            nb = 1 - b

            @pl.when(c + 1 < chunks)
            def _prefetch():
                nbase = base0 + (c + 1) * C
                pltpu.async_copy(x_hbm.at[pl.ds(nbase, C)], idx_v.at[nb],
                                 sem_idx.at[nb])

            pltpu.make_async_copy(x_hbm.at[pl.ds(base0, C)], idx_v.at[b],
                                  sem_idx.at[b]).wait()

            @pl.when(c >= 2)
            def _drain():
                obase = (base0 + (c - 2) * C) * EMBED
                pltpu.make_async_copy(rows_v.at[b],
                                      out_hbm.at[pl.ds(obase, C * EMBED)],
                                      sem_out.at[b]).wait()

            rows_b = rows_v.at[b]

            @plsc.parallel_loop(0, C // 16)
            def _i(i):
                idx16 = idx_v[b, pl.ds(i * 16, 16)] * EMBED
                for l in range(16):
                    src = idx16[l]
                    dst = (i * 16 + l) * EMBED
                    for k in range(EMBED // 16):
                        rows_b[pl.ds(dst + k * 16, 16)] = (
                            table_v[pl.ds(src + k * 16, 16)])

            obase = (base0 + c * C) * EMBED
            pltpu.async_copy(rows_b, out_hbm.at[pl.ds(obase, C * EMBED)],
                             sem_out.at[b])

        @pl.loop(0, 2)
        def _tail(t):
            c = chunks - 2 + t
            b = c % 2
            obase = (base0 + c * C) * EMBED
            pltpu.make_async_copy(rows_v.at[b],
                                  out_hbm.at[pl.ds(obase, C * EMBED)],
                                  sem_out.at[b]).wait()

    return sc_lookup


def kernel(x, table):
    lead = x.shape[:-1]
    k = x.shape[-1]
    B = 1
    for s in x.shape:
        B *= s
    xf = x.reshape(B).astype(jnp.int32)
    out = _make_sc_lookup(B, 512)(xf, table.reshape(-1))
    return out.reshape(lead + (k * EMBED,))
